# Initial kernel scaffold; baseline (speedup 1.0000x reference)
import jax
import jax.numpy as jnp
from jax.experimental import pallas as pl


def kernel(x, emb):
    raise NotImplementedError("write your pallas kernel here")


# speedup vs baseline: 3.5327x; 3.5327x over previous
import jax
import jax.numpy as jnp
from jax import lax
from jax.experimental import pallas as pl
from jax.experimental.pallas import tpu as pltpu
from jax.experimental.pallas import tpu_sc as plsc

_NV = 100000
_ED = 64
_N_CONT = 4
_NSLOT = 13
_B, _L, _C = 4096, 50, 17
_OUT_D = _N_CONT + _NSLOT * _ED
_SLOT_STRIDE = 56
_IDS_PER_B = _NSLOT * _SLOT_STRIDE
_NVEC = _NSLOT * (_ED // 16)

_NC, _NS = 2, 16
_NW = _NC * _NS
_B_PER_W = _B // _NW
_W0, _W1 = 24, 26

_DNUMS = lax.GatherDimensionNumbers(
    offset_dims=(), collapsed_slice_dims=(0,), start_index_map=(0,))


def _rot12(v, perm):
    return lax.gather(v, perm[:, None], _DNUMS, (1,),
                      mode=lax.GatherScatterMode.PROMISE_IN_BOUNDS)


def _emb_body(ids_hbm, cont_hbm, emb_hbm, out_hbm, idx_v, rows_v, buf_v,
              cont_v, sem):
    wid = lax.axis_index("s") * _NC + lax.axis_index("c")
    lane = lax.iota(jnp.int32, 16)
    perm = jnp.bitwise_and(lane + 12, 15)
    head = lane < _N_CONT

    def batch_body(bl, carry):
        b = wid * _B_PER_W + bl
        pltpu.sync_copy(ids_hbm.at[pl.ds(b * _IDS_PER_B, _IDS_PER_B)], idx_v)
        pltpu.sync_copy(cont_hbm.at[pl.ds(b * _L * 16, _L * 16)], cont_v)
        for r0, sz in ((0, _W0), (_W0, _W1)):
            cps = []
            for j in range(_NSLOT):
                src = emb_hbm.at[idx_v.at[pl.ds(j * _SLOT_STRIDE + r0, sz)]]
                cps.append(pltpu.async_copy(src, rows_v.at[j, pl.ds(0, sz)], sem))
            for cp in cps:
                cp.wait()

            def row_body(rl, c2):
                r = r0 + rl
                prev = cont_v[pl.ds(r * 16, 16)]
                for t in range(_NVEC):
                    s = rows_v[t // 4, rl, pl.ds(16 * (t % 4), 16)]
                    ps = _rot12(s, perm)
                    buf_v[rl, pl.ds(16 * t, 16)] = jnp.where(head, prev, ps)
                    prev = ps
                buf_v.at[rl][pl.ds(16 * _NVEC, _N_CONT)] = prev[0:_N_CONT]
                return c2

            lax.fori_loop(0, sz, row_body, 0)
            pltpu.sync_copy(buf_v.at[pl.ds(0, sz)],
                            out_hbm.at[b, pl.ds(r0, sz)])
        return carry

    lax.fori_loop(0, _B_PER_W, batch_body, 0)


@jax.jit
def kernel(x, emb):
    xf = x.reshape(_B * _L, _C)
    ids = jnp.clip(xf[:, _N_CONT:_C].astype(jnp.int32), 0, None)
    ids_b = ids.reshape(_B, _L, _NSLOT).transpose(0, 2, 1)
    ids_p = jnp.pad(ids_b, ((0, 0), (0, 0), (0, _SLOT_STRIDE - _L)))
    ids1 = ids_p.reshape(-1)
    cont16 = jnp.pad(xf[:, :_N_CONT], ((0, 0), (0, 16 - _N_CONT))).reshape(-1)
    emb128 = jnp.pad(emb, ((0, 0), (0, 128 - _ED)))

    mesh = plsc.VectorSubcoreMesh(core_axis_name="c", subcore_axis_name="s")
    out = pl.kernel(
        _emb_body,
        mesh=mesh,
        out_type=jax.ShapeDtypeStruct((_B, _L, _OUT_D), jnp.float32),
        scratch_types=[
            pltpu.VMEM((_IDS_PER_B,), jnp.int32),
            pltpu.VMEM((_NSLOT, _W1, 128), jnp.float32),
            pltpu.VMEM((_W1, _OUT_D), jnp.float32),
            pltpu.VMEM((_L * 16,), jnp.float32),
            pltpu.SemaphoreType.DMA,
        ],
    )(ids1, cont16, emb128)
    return out


# speedup vs baseline: 4.1532x; 1.1757x over previous
import jax
import jax.numpy as jnp
from jax import lax
from jax.experimental import pallas as pl
from jax.experimental.pallas import tpu as pltpu
from jax.experimental.pallas import tpu_sc as plsc

_NV = 100000
_ED = 64
_N_CONT = 4
_NSLOT = 13
_B, _L, _C = 4096, 50, 17
_OUT_D = _N_CONT + _NSLOT * _ED

_NC, _NS = 2, 16
_NW = _NC * _NS
_B_PER_W = _B // _NW
_GB = 4
_XG = _GB * _L * _C
_NGRP = _B_PER_W // _GB
_WPB = 3
_NWAVE = _B_PER_W * _WPB
_W2 = 18
_G16 = _NSLOT * 16
_G18 = 240
_IHALF = 320

_DNUMS = lax.GatherDimensionNumbers(
    offset_dims=(), collapsed_slice_dims=(0,), start_index_map=(0,))


def _rot12(v, perm):
    return lax.gather(v, perm[:, None], _DNUMS, (1,),
                      mode=lax.GatherScatterMode.PROMISE_IN_BOUNDS)


def _emb_body(x_hbm, emb_hbm, out_hbm, x_v, idx_v, rows_v, buf_v,
              xsem0, xsem1, gsem0, gsem1, osem0, osem1):
    wid = lax.axis_index("s") * _NC + lax.axis_index("c")
    lane = lax.iota(jnp.int32, 16)
    lane17 = lane * 17
    perm = jnp.bitwise_and(lane + 12, 15)
    head = lane < _N_CONT
    zeros16 = jnp.zeros((16,), jnp.int32)

    def extract(m):
        bb = (m // _WPB) % _GB
        w = m % _WPB
        xbase = ((m // (_WPB * _GB)) % 2) * _XG
        base0 = xbase + 850 * bb + 4
        ibase = (m % 2) * _IHALF

        def ext16(_):
            for j in range(_NSLOT):
                pos = lane17 + (base0 + 17 * (16 * w) + j)
                g = plsc.load_gather(x_v, [pos])
                ids = jnp.maximum(g.astype(jnp.int32), 0)
                idx_v[pl.ds(ibase + 16 * j, 16)] = ids
            return 0

        def ext18(_):
            for j in range(_NSLOT):
                pos = lane17 + (base0 + 17 * 32 + j)
                g = plsc.load_gather(x_v, [pos])
                ids = jnp.maximum(g.astype(jnp.int32), 0)
                plsc.store_scatter(idx_v, [ibase + 18 * j + lane], ids)
                post = jnp.minimum(lane17 + (base0 + 17 * 48 + j),
                                   xbase + _XG - 1)
                g2 = plsc.load_gather(x_v, [post])
                ids2 = jnp.maximum(g2.astype(jnp.int32), 0)
                plsc.store_scatter(idx_v, [ibase + 18 * j + 16 + lane], ids2,
                                   mask=lane < 2)
            plsc.store_scatter(idx_v, [ibase + 234 + lane], zeros16,
                               mask=lane < 6)
            return 0

        lax.cond(w == 2, ext18, ext16, 0)

    def batch_of(k):
        return wid * _B_PER_W + k // _WPB

    def wave_body(k, carry):
        m = k + 1
        @pl.when(jnp.logical_and(k % (_WPB * _GB) == 0,
                                 k // (_WPB * _GB) + 1 < _NGRP))
        def _():
            g = k // (_WPB * _GB) + 1
            src = x_hbm.at[pl.ds(pl.multiple_of((wid * _NGRP + g) * _XG, 8),
                                 _XG)]
            dst = x_v.at[pl.ds(pl.multiple_of((g % 2) * _XG, 8), _XG)]

            @pl.when(g % 2 == 0)
            def _():
                pltpu.async_copy(src, dst, xsem0)

            @pl.when(g % 2 == 1)
            def _():
                pltpu.async_copy(src, dst, xsem1)

        @pl.when(m < _NWAVE)
        def _():
            @pl.when(jnp.logical_and(m % (_WPB * _GB) == 0, m > 0))
            def _():
                g = m // (_WPB * _GB)
                src = x_hbm.at[pl.ds(pl.multiple_of((wid * _NGRP + g) * _XG,
                                                    8), _XG)]
                dst = x_v.at[pl.ds(pl.multiple_of((g % 2) * _XG, 8), _XG)]

                @pl.when(g % 2 == 0)
                def _():
                    pltpu.make_async_copy(src, dst, xsem0).wait()

                @pl.when(g % 2 == 1)
                def _():
                    pltpu.make_async_copy(src, dst, xsem1).wait()

            extract(m)
            ibase = (m % 2) * _IHALF
            rbase = pl.multiple_of((m % 2) * 240, 8)

            @pl.when(m % _WPB != 2)
            def _():
                src = emb_hbm.at[idx_v.at[pl.ds(ibase, _G16)]]
                dst = rows_v.at[pl.ds(rbase, _G16)]

                @pl.when(m % 2 == 0)
                def _():
                    pltpu.async_copy(src, dst, gsem0)

                @pl.when(m % 2 == 1)
                def _():
                    pltpu.async_copy(src, dst, gsem1)

            @pl.when(m % _WPB == 2)
            def _():
                src = emb_hbm.at[idx_v.at[pl.ds(ibase, _G18)]]
                dst = rows_v.at[pl.ds(rbase, _G18)]

                @pl.when(m % 2 == 0)
                def _():
                    pltpu.async_copy(src, dst, gsem0)

                @pl.when(m % 2 == 1)
                def _():
                    pltpu.async_copy(src, dst, gsem1)

        kbase = (k % 2) * _IHALF
        krbase = pl.multiple_of((k % 2) * 240, 8)
        for par in (0, 1):
            gs = [gsem0, gsem1][par]

            @pl.when(jnp.logical_and(k % 2 == par, k % _WPB != 2))
            def _():
                src = emb_hbm.at[idx_v.at[pl.ds(kbase, _G16)]]
                dst = rows_v.at[pl.ds(krbase, _G16)]
                pltpu.make_async_copy(src, dst, gs).wait()

            @pl.when(jnp.logical_and(k % 2 == par, k % _WPB == 2))
            def _():
                src = emb_hbm.at[idx_v.at[pl.ds(kbase, _G18)]]
                dst = rows_v.at[pl.ds(krbase, _G18)]
                pltpu.make_async_copy(src, dst, gs).wait()

        @pl.when(k >= 2)
        def _():
            kk = k - 2
            b2 = batch_of(kk)
            r02 = 16 * (kk % _WPB)
            for par in (0, 1):
                osp = [osem0, osem1][par]

                @pl.when(jnp.logical_and(kk % 2 == par, kk % _WPB != 2))
                def _():
                    src = buf_v.at[par, pl.ds(0, 16)]
                    dst = out_hbm.at[b2, pl.ds(r02, 16)]
                    pltpu.make_async_copy(src, dst, osp).wait()

                @pl.when(jnp.logical_and(kk % 2 == par, kk % _WPB == 2))
                def _():
                    src = buf_v.at[par]
                    dst = out_hbm.at[b2, pl.ds(32, _W2)]
                    pltpu.make_async_copy(src, dst, osp).wait()

        b = batch_of(k)
        w = k % _WPB
        r0 = 16 * w
        sz = jnp.where(w == 2, _W2, 16)
        bb = (k // _WPB) % _GB
        xbase = ((k // (_WPB * _GB)) % 2) * _XG
        cbase = xbase + 850 * bb
        bset = k % 2
        stride = jnp.where(w == 2, 18, 16)
        tail_col = lane + (16 * 52)

        def row_body(rl, c2):
            r = r0 + rl
            cont = plsc.load_gather(x_v, [lane + (cbase + 17 * r)])
            prev = cont
            rowrow = jnp.full((16,), rl, jnp.int32)
            for t in range(52):
                j, kk4 = t // 4, t % 4
                srow = krbase + stride * j + rl
                s = rows_v[srow, pl.ds(16 * kk4, 16)]
                ps = _rot12(s, perm)
                buf_v[bset, rl, pl.ds(16 * t, 16)] = jnp.where(head, prev, ps)
                prev = ps
            plsc.store_scatter(buf_v.at[bset], [rowrow, tail_col], prev,
                               mask=head)
            return c2

        lax.fori_loop(0, sz, row_body, 0)

        for par in (0, 1):
            osp = [osem0, osem1][par]

            @pl.when(jnp.logical_and(k % 2 == par, k % _WPB != 2))
            def _():
                pltpu.async_copy(buf_v.at[par, pl.ds(0, 16)],
                                 out_hbm.at[b, pl.ds(r0, 16)], osp)

            @pl.when(jnp.logical_and(k % 2 == par, k % _WPB == 2))
            def _():
                pltpu.async_copy(buf_v.at[par],
                                 out_hbm.at[b, pl.ds(32, _W2)], osp)

        return carry

    src0 = x_hbm.at[pl.ds(pl.multiple_of(wid * _NGRP * _XG, 8), _XG)]
    pltpu.sync_copy(src0, x_v.at[pl.ds(0, _XG)])
    extract(0)
    src = emb_hbm.at[idx_v.at[pl.ds(0, _G16)]]
    pltpu.async_copy(src, rows_v.at[pl.ds(0, _G16)], gsem0)

    lax.fori_loop(0, _NWAVE, wave_body, 0)

    for kk in (_NWAVE - 2, _NWAVE - 1):
        b2 = wid * _B_PER_W + kk // _WPB
        par = kk % 2
        osp = [osem0, osem1][par]
        if kk % _WPB != 2:
            pltpu.make_async_copy(buf_v.at[par, pl.ds(0, 16)],
                                  out_hbm.at[b2, pl.ds(16 * (kk % _WPB), 16)],
                                  osp).wait()
        else:
            pltpu.make_async_copy(buf_v.at[par],
                                  out_hbm.at[b2, pl.ds(32, _W2)], osp).wait()


@jax.jit
def kernel(x, emb):
    x1 = x.reshape(-1)
    emb128 = jnp.pad(emb, ((0, 0), (0, 128 - _ED)))

    mesh = plsc.VectorSubcoreMesh(core_axis_name="c", subcore_axis_name="s")
    out = pl.kernel(
        _emb_body,
        mesh=mesh,
        compiler_params=pltpu.CompilerParams(needs_layout_passes=False),
        out_type=jax.ShapeDtypeStruct((_B, _L, _OUT_D), jnp.float32),
        scratch_types=[
            pltpu.VMEM((2 * _XG,), jnp.float32),
            pltpu.VMEM((2 * _IHALF,), jnp.int32),
            pltpu.VMEM((2 * 240, 128), jnp.float32),
            pltpu.VMEM((2, _W2, _OUT_D), jnp.float32),
            pltpu.SemaphoreType.DMA,
            pltpu.SemaphoreType.DMA,
            pltpu.SemaphoreType.DMA,
            pltpu.SemaphoreType.DMA,
            pltpu.SemaphoreType.DMA,
            pltpu.SemaphoreType.DMA,
        ],
    )(x1, emb128)
    return out


# speedup vs baseline: 4.1546x; 1.0003x over previous
import jax
import jax.numpy as jnp
from jax import lax
from jax.experimental import pallas as pl
from jax.experimental.pallas import tpu as pltpu
from jax.experimental.pallas import tpu_sc as plsc

_NV = 100000
_ED = 64
_N_CONT = 4
_NSLOT = 13
_B, _L, _C = 4096, 50, 17
_OUT_D = _N_CONT + _NSLOT * _ED

_NC, _NS = 2, 16
_NW = _NC * _NS
_B_PER_W = _B // _NW
_GB = 4
_XG = _GB * _L * _C
_NGRP = _B_PER_W // _GB
_WPB = 3
_NWAVE = _B_PER_W * _WPB
_W2 = 18
_G16 = _NSLOT * 16
_G18 = 240
_IHALF = 320

_DNUMS = lax.GatherDimensionNumbers(
    offset_dims=(), collapsed_slice_dims=(0,), start_index_map=(0,))


def _rot12(v, perm):
    return lax.gather(v, perm[:, None], _DNUMS, (1,),
                      mode=lax.GatherScatterMode.PROMISE_IN_BOUNDS)


def _emb_body(x_hbm, emb_hbm, out_hbm, x_v, idx_v, rows_v, buf_v,
              xsem0, xsem1, gsem0, gsem1, osem0, osem1):
    wid = lax.axis_index("s") * _NC + lax.axis_index("c")
    lane = lax.iota(jnp.int32, 16)
    lane17 = lane * 17
    perm = jnp.bitwise_and(lane + 12, 15)
    head = lane < _N_CONT
    zeros16 = jnp.zeros((16,), jnp.int32)

    def extract(m):
        bb = (m // _WPB) % _GB
        w = m % _WPB
        xbase = ((m // (_WPB * _GB)) % 2) * _XG
        base0 = xbase + 850 * bb + 4
        ibase = (m % 2) * _IHALF

        def ext16(_):
            for j in range(_NSLOT):
                pos = lane17 + (base0 + 17 * (16 * w) + j)
                g = plsc.load_gather(x_v, [pos])
                ids = jnp.maximum(g.astype(jnp.int32), 0)
                idx_v[pl.ds(ibase + 16 * j, 16)] = ids
            return 0

        def ext18(_):
            for j in range(_NSLOT):
                pos = lane17 + (base0 + 17 * 32 + j)
                g = plsc.load_gather(x_v, [pos])
                ids = jnp.maximum(g.astype(jnp.int32), 0)
                plsc.store_scatter(idx_v, [ibase + 18 * j + lane], ids)
                post = jnp.minimum(lane17 + (base0 + 17 * 48 + j),
                                   xbase + _XG - 1)
                g2 = plsc.load_gather(x_v, [post])
                ids2 = jnp.maximum(g2.astype(jnp.int32), 0)
                plsc.store_scatter(idx_v, [ibase + 18 * j + 16 + lane], ids2,
                                   mask=lane < 2)
            plsc.store_scatter(idx_v, [ibase + 234 + lane], zeros16,
                               mask=lane < 6)
            return 0

        lax.cond(w == 2, ext18, ext16, 0)

    def batch_of(k):
        return wid * _B_PER_W + k // _WPB

    def wave_body(k, carry):
        m = k + 1
        @pl.when(jnp.logical_and(k % (_WPB * _GB) == 0,
                                 k // (_WPB * _GB) + 1 < _NGRP))
        def _():
            g = k // (_WPB * _GB) + 1
            src = x_hbm.at[pl.ds(pl.multiple_of((wid * _NGRP + g) * _XG, 8),
                                 _XG)]
            dst = x_v.at[pl.ds(pl.multiple_of((g % 2) * _XG, 8), _XG)]

            @pl.when(g % 2 == 0)
            def _():
                pltpu.async_copy(src, dst, xsem0)

            @pl.when(g % 2 == 1)
            def _():
                pltpu.async_copy(src, dst, xsem1)

        @pl.when(m < _NWAVE)
        def _():
            @pl.when(jnp.logical_and(m % (_WPB * _GB) == 0, m > 0))
            def _():
                g = m // (_WPB * _GB)
                src = x_hbm.at[pl.ds(pl.multiple_of((wid * _NGRP + g) * _XG,
                                                    8), _XG)]
                dst = x_v.at[pl.ds(pl.multiple_of((g % 2) * _XG, 8), _XG)]

                @pl.when(g % 2 == 0)
                def _():
                    pltpu.make_async_copy(src, dst, xsem0).wait()

                @pl.when(g % 2 == 1)
                def _():
                    pltpu.make_async_copy(src, dst, xsem1).wait()

            extract(m)
            ibase = (m % 2) * _IHALF
            rbase = pl.multiple_of((m % 2) * 240, 8)

            @pl.when(m % _WPB != 2)
            def _():
                src = emb_hbm.at[idx_v.at[pl.ds(ibase, _G16)]]
                dst = rows_v.at[pl.ds(rbase, _G16)]

                @pl.when(m % 2 == 0)
                def _():
                    pltpu.async_copy(src, dst, gsem0)

                @pl.when(m % 2 == 1)
                def _():
                    pltpu.async_copy(src, dst, gsem1)

            @pl.when(m % _WPB == 2)
            def _():
                src = emb_hbm.at[idx_v.at[pl.ds(ibase, _G18)]]
                dst = rows_v.at[pl.ds(rbase, _G18)]

                @pl.when(m % 2 == 0)
                def _():
                    pltpu.async_copy(src, dst, gsem0)

                @pl.when(m % 2 == 1)
                def _():
                    pltpu.async_copy(src, dst, gsem1)

        kbase = (k % 2) * _IHALF
        krbase = pl.multiple_of((k % 2) * 240, 8)
        for par in (0, 1):
            gs = [gsem0, gsem1][par]

            @pl.when(jnp.logical_and(k % 2 == par, k % _WPB != 2))
            def _():
                src = emb_hbm.at[idx_v.at[pl.ds(kbase, _G16)]]
                dst = rows_v.at[pl.ds(krbase, _G16)]
                pltpu.make_async_copy(src, dst, gs).wait()

            @pl.when(jnp.logical_and(k % 2 == par, k % _WPB == 2))
            def _():
                src = emb_hbm.at[idx_v.at[pl.ds(kbase, _G18)]]
                dst = rows_v.at[pl.ds(krbase, _G18)]
                pltpu.make_async_copy(src, dst, gs).wait()

        @pl.when(k >= 2)
        def _():
            kk = k - 2
            b2 = batch_of(kk)
            r02 = 16 * (kk % _WPB)
            for par in (0, 1):
                osp = [osem0, osem1][par]

                @pl.when(jnp.logical_and(kk % 2 == par, kk % _WPB != 2))
                def _():
                    src = buf_v.at[par, pl.ds(0, 16)]
                    dst = out_hbm.at[b2, pl.ds(r02, 16)]
                    pltpu.make_async_copy(src, dst, osp).wait()

                @pl.when(jnp.logical_and(kk % 2 == par, kk % _WPB == 2))
                def _():
                    src = buf_v.at[par]
                    dst = out_hbm.at[b2, pl.ds(32, _W2)]
                    pltpu.make_async_copy(src, dst, osp).wait()

        b = batch_of(k)
        w = k % _WPB
        r0 = 16 * w
        bb = (k // _WPB) % _GB
        xbase = ((k // (_WPB * _GB)) % 2) * _XG
        cbase = xbase + 850 * bb
        bset = k % 2
        tail_col = lane + (16 * 52)

        def make_asm(stride_c, sz_c):
            def asm(_):
                def row_body(rl, c2):
                    r = r0 + rl
                    cont = plsc.load_gather(x_v, [lane + (cbase + 17 * r)])
                    prev = cont
                    rowrow = jnp.full((16,), rl, jnp.int32)
                    for t in range(52):
                        j, kk4 = t // 4, t % 4
                        srow = krbase + rl + stride_c * j
                        s = rows_v[srow, pl.ds(16 * kk4, 16)]
                        ps = _rot12(s, perm)
                        buf_v[bset, rl, pl.ds(16 * t, 16)] = jnp.where(
                            head, prev, ps)
                        prev = ps
                    plsc.store_scatter(buf_v.at[bset], [rowrow, tail_col],
                                       prev, mask=head)
                    return c2

                lax.fori_loop(0, sz_c, row_body, 0)
                return 0
            return asm

        lax.cond(w == 2, make_asm(18, _W2), make_asm(16, 16), 0)

        for par in (0, 1):
            osp = [osem0, osem1][par]

            @pl.when(jnp.logical_and(k % 2 == par, k % _WPB != 2))
            def _():
                pltpu.async_copy(buf_v.at[par, pl.ds(0, 16)],
                                 out_hbm.at[b, pl.ds(r0, 16)], osp)

            @pl.when(jnp.logical_and(k % 2 == par, k % _WPB == 2))
            def _():
                pltpu.async_copy(buf_v.at[par],
                                 out_hbm.at[b, pl.ds(32, _W2)], osp)

        return carry

    src0 = x_hbm.at[pl.ds(pl.multiple_of(wid * _NGRP * _XG, 8), _XG)]
    pltpu.sync_copy(src0, x_v.at[pl.ds(0, _XG)])
    extract(0)
    src = emb_hbm.at[idx_v.at[pl.ds(0, _G16)]]
    pltpu.async_copy(src, rows_v.at[pl.ds(0, _G16)], gsem0)

    lax.fori_loop(0, _NWAVE, wave_body, 0)

    for kk in (_NWAVE - 2, _NWAVE - 1):
        b2 = wid * _B_PER_W + kk // _WPB
        par = kk % 2
        osp = [osem0, osem1][par]
        if kk % _WPB != 2:
            pltpu.make_async_copy(buf_v.at[par, pl.ds(0, 16)],
                                  out_hbm.at[b2, pl.ds(16 * (kk % _WPB), 16)],
                                  osp).wait()
        else:
            pltpu.make_async_copy(buf_v.at[par],
                                  out_hbm.at[b2, pl.ds(32, _W2)], osp).wait()


@jax.jit
def kernel(x, emb):
    x1 = x.reshape(-1)
    emb128 = jnp.pad(emb, ((0, 0), (0, 128 - _ED)))

    mesh = plsc.VectorSubcoreMesh(core_axis_name="c", subcore_axis_name="s")
    out = pl.kernel(
        _emb_body,
        mesh=mesh,
        compiler_params=pltpu.CompilerParams(needs_layout_passes=False),
        out_type=jax.ShapeDtypeStruct((_B, _L, _OUT_D), jnp.float32),
        scratch_types=[
            pltpu.VMEM((2 * _XG,), jnp.float32),
            pltpu.VMEM((2 * _IHALF,), jnp.int32),
            pltpu.VMEM((2 * 240, 128), jnp.float32),
            pltpu.VMEM((2, _W2, _OUT_D), jnp.float32),
            pltpu.SemaphoreType.DMA,
            pltpu.SemaphoreType.DMA,
            pltpu.SemaphoreType.DMA,
            pltpu.SemaphoreType.DMA,
            pltpu.SemaphoreType.DMA,
            pltpu.SemaphoreType.DMA,
        ],
    )(x1, emb128)
    return out
